# baseline (speedup 1.0000x reference)
import functools

import jax
import jax.numpy as jnp
from jax.experimental import pallas as pl
from jax.experimental.pallas import tpu as pltpu

_LEAKY = 0.5


def _pass1_body(adj_ref, iemb_ref, uemb_ref, ti_ref, tu_ref):
    r = pl.program_id(0)

    @pl.when(r == 0)
    def _init():
        tu_ref[...] = jnp.zeros_like(tu_ref)

    adj = adj_ref[...].astype(jnp.bfloat16)
    ti_ref[...] = jnp.dot(adj, iemb_ref[...].astype(jnp.bfloat16),
                          preferred_element_type=jnp.float32)
    tu_ref[...] += jax.lax.dot_general(
        adj, uemb_ref[...].astype(jnp.bfloat16), (((0,), (0,)), ((), ())),
        preferred_element_type=jnp.float32)


def _pass2_body(adj_ref, tu_ref, ti_ref, hu_ref, hi_ref, *, nsteps):
    r = pl.program_id(0)

    @pl.when(r == 0)
    def _init():
        hi_ref[...] = jnp.zeros_like(hi_ref)

    adj = adj_ref[...].astype(jnp.bfloat16)
    hu = jnp.dot(adj, tu_ref[...].astype(jnp.bfloat16),
                 preferred_element_type=jnp.float32)
    hu_ref[...] = jnp.where(hu >= 0, hu, _LEAKY * hu)
    hi_ref[...] += jax.lax.dot_general(
        adj, ti_ref[...].astype(jnp.bfloat16), (((0,), (0,)), ((), ())),
        preferred_element_type=jnp.float32)

    @pl.when(r == nsteps - 1)
    def _act():
        hi = hi_ref[...]
        hi_ref[...] = jnp.where(hi >= 0, hi, _LEAKY * hi)


@functools.partial(jax.jit, static_argnames=("stripe",))
def _dhcf(adj, user_emb, item_emb, stripe=256):
    n_u, n_i = adj.shape
    d = user_emb.shape[1]
    nsteps = n_u // stripe

    grid = (nsteps,)
    params = pltpu.CompilerParams(dimension_semantics=("arbitrary",))

    t_i, t_u = pl.pallas_call(
        _pass1_body,
        grid=grid,
        in_specs=[
            pl.BlockSpec((stripe, n_i), lambda r: (r, 0)),
            pl.BlockSpec((n_i, d), lambda r: (0, 0)),
            pl.BlockSpec((stripe, d), lambda r: (r, 0)),
        ],
        out_specs=[
            pl.BlockSpec((stripe, d), lambda r: (r, 0)),
            pl.BlockSpec((n_i, d), lambda r: (0, 0)),
        ],
        out_shape=[
            jax.ShapeDtypeStruct((n_u, d), jnp.float32),
            jax.ShapeDtypeStruct((n_i, d), jnp.float32),
        ],
        compiler_params=params,
    )(adj, item_emb, user_emb)

    h_u, h_i = pl.pallas_call(
        functools.partial(_pass2_body, nsteps=nsteps),
        grid=grid,
        in_specs=[
            pl.BlockSpec((stripe, n_i), lambda r: (r, 0)),
            pl.BlockSpec((n_i, d), lambda r: (0, 0)),
            pl.BlockSpec((stripe, d), lambda r: (r, 0)),
        ],
        out_specs=[
            pl.BlockSpec((stripe, d), lambda r: (r, 0)),
            pl.BlockSpec((n_i, d), lambda r: (0, 0)),
        ],
        out_shape=[
            jax.ShapeDtypeStruct((n_u, d), jnp.float32),
            jax.ShapeDtypeStruct((n_i, d), jnp.float32),
        ],
        compiler_params=params,
    )(adj, t_u, t_i)

    user_all = jnp.concatenate([user_emb, h_u, h_u], axis=1)
    item_all = jnp.concatenate([item_emb, h_i, h_i], axis=1)
    return user_all, item_all


def kernel(adj, user_emb, item_emb):
    return _dhcf(adj, user_emb, item_emb)


# speedup vs baseline: 1.2001x; 1.2001x over previous
import functools

import jax
import jax.numpy as jnp
from jax.experimental import pallas as pl
from jax.experimental.pallas import tpu as pltpu

_LEAKY = 0.5


def _pass1_body(adj_ref, iemb_ref, uembT_ref, ti_ref, tuT_ref, hiT_ref, adj8_ref,
                *, nsteps):
    r = pl.program_id(0)

    @pl.when(r == 0)
    def _init():
        tuT_ref[...] = jnp.zeros_like(tuT_ref)
        hiT_ref[...] = jnp.zeros_like(hiT_ref)

    adj = adj_ref[...]
    adjb = adj.astype(jnp.bfloat16)
    adj8_ref[...] = adj.astype(jnp.int8)

    ti = jnp.dot(adjb, iemb_ref[...].astype(jnp.bfloat16),
                 preferred_element_type=jnp.float32)
    ti_ref[...] = ti
    tuT_ref[...] += jnp.dot(uembT_ref[...].astype(jnp.bfloat16), adjb,
                            preferred_element_type=jnp.float32)
    hiT_ref[...] += jnp.dot(ti.astype(jnp.bfloat16).T, adjb,
                            preferred_element_type=jnp.float32)

    @pl.when(r == nsteps - 1)
    def _act():
        hi = hiT_ref[...]
        hiT_ref[...] = jnp.where(hi >= 0, hi, _LEAKY * hi)


def _pass2_body(adj8_ref, tu_ref, hu_ref):
    hu = jnp.dot(adj8_ref[...].astype(jnp.bfloat16),
                 tu_ref[...].astype(jnp.bfloat16),
                 preferred_element_type=jnp.float32)
    hu_ref[...] = jnp.where(hu >= 0, hu, _LEAKY * hu)


@functools.partial(jax.jit, static_argnames=("stripe",))
def _dhcf(adj, user_emb, item_emb, stripe=256):
    n_u, n_i = adj.shape
    d = user_emb.shape[1]
    nsteps = n_u // stripe

    grid = (nsteps,)
    params = pltpu.CompilerParams(dimension_semantics=("arbitrary",))

    t_i, t_uT, h_iT, adj8 = pl.pallas_call(
        functools.partial(_pass1_body, nsteps=nsteps),
        grid=grid,
        in_specs=[
            pl.BlockSpec((stripe, n_i), lambda r: (r, 0)),
            pl.BlockSpec((n_i, d), lambda r: (0, 0)),
            pl.BlockSpec((d, stripe), lambda r: (0, r)),
        ],
        out_specs=[
            pl.BlockSpec((stripe, d), lambda r: (r, 0)),
            pl.BlockSpec((d, n_i), lambda r: (0, 0)),
            pl.BlockSpec((d, n_i), lambda r: (0, 0)),
            pl.BlockSpec((stripe, n_i), lambda r: (r, 0)),
        ],
        out_shape=[
            jax.ShapeDtypeStruct((n_u, d), jnp.float32),
            jax.ShapeDtypeStruct((d, n_i), jnp.float32),
            jax.ShapeDtypeStruct((d, n_i), jnp.float32),
            jax.ShapeDtypeStruct((n_u, n_i), jnp.int8),
        ],
        compiler_params=params,
    )(adj, item_emb, user_emb.T)

    h_u = pl.pallas_call(
        _pass2_body,
        grid=grid,
        in_specs=[
            pl.BlockSpec((stripe, n_i), lambda r: (r, 0)),
            pl.BlockSpec((n_i, d), lambda r: (0, 0)),
        ],
        out_specs=pl.BlockSpec((stripe, d), lambda r: (r, 0)),
        out_shape=jax.ShapeDtypeStruct((n_u, d), jnp.float32),
        compiler_params=params,
    )(adj8, t_uT.T)

    h_i = h_iT.T
    user_all = jnp.concatenate([user_emb, h_u, h_u], axis=1)
    item_all = jnp.concatenate([item_emb, h_i, h_i], axis=1)
    return user_all, item_all


def kernel(adj, user_emb, item_emb):
    return _dhcf(adj, user_emb, item_emb)
